# baseline (speedup 1.0000x reference)
import functools

import ml_dtypes
import numpy as np
import jax
import jax.numpy as jnp
from jax import lax
from jax.experimental import pallas as pl
from jax.experimental.pallas import tpu as pltpu


def _build_g1() -> np.ndarray:
    g = np.zeros((25, 112, 256), np.float32)
    qq = np.arange(2)[:, None]
    jp = np.arange(14)[None, :]
    for half in range(2):
        for eo in range(2):
            p = (56 * half + 28 * eo + 14 * qq + jp).ravel()
            q = 2 * half + qq
            for di in range(5):
                for dj in range(5):
                    k = (32 * (q + di) + 2 * jp + eo + dj).ravel()
                    g[5 * di + dj, p, k] = 1.0
    return g


def _build_g2() -> np.ndarray:
    g = np.zeros((150, 20, 576), np.float32)
    rr = np.arange(2)[:, None]
    j = np.arange(10)[None, :]
    p = (2 * j + rr).ravel()
    for di in range(5):
        for dj in range(5):
            for ci in range(6):
                k = (96 * (rr + di) + 6 * (j + dj) + ci).ravel()
                g[(5 * di + dj) * 6 + ci, p, k] = 1.0
    return g


_BF16 = np.dtype(ml_dtypes.bfloat16)
_G1 = _build_g1().astype(_BF16)
_G2 = _build_g2().astype(_BF16)


def _lenet_body(x_ref, t1_ref, b1_ref, t2_ref, b2_ref,
                wf1_ref, bf1_ref, wf2_ref, bf2_ref, wf3_ref, bf3_ref,
                o_ref, p1_ref, p2_ref):
    f32 = jnp.float32
    bf16 = jnp.bfloat16
    TB = x_ref.shape[1]
    NC = 256
    zpad = jnp.zeros((8, NC), bf16)

    @pl.loop(0, TB // NC)
    def _chunk(nb):
        sl = pl.ds(NC * nb, NC)

        t1 = t1_ref[...]
        b1b = b1_ref[...]
        for g in range(7):
            slab = x_ref[pl.ds(128 * g, 256), sl].astype(bf16)
            acc = lax.dot_general(t1, slab, (((1,), (0,)), ((), ())),
                                  preferred_element_type=f32)
            a = acc.reshape(2, 4, 88, NC)
            for half in range(2):
                b = a[half]
                pooled = jnp.maximum(jnp.maximum(b[0], b[1]),
                                     jnp.maximum(b[2], b[3]))
                pooled = jnp.maximum(pooled + b1b, 0.0)
                ri = 2 * g + half
                p1_ref[pl.ds(96 * ri, 96), :] = jnp.concatenate(
                    [pooled.astype(bf16), zpad], axis=0)

        t2 = t2_ref[...]
        b2b = b2_ref[...].reshape(1, 16, 1)
        for pr in range(5):
            slab2 = p1_ref[pl.ds(192 * pr, 576), :]
            acc2 = lax.dot_general(t2, slab2, (((1,), (0,)), ((), ())),
                                   preferred_element_type=f32)
            a2 = acc2.reshape(5, 2, 2, 16, NC)
            w2 = jnp.maximum(jnp.maximum(a2[:, 0, 0], a2[:, 0, 1]),
                             jnp.maximum(a2[:, 1, 0], a2[:, 1, 1]))
            w2 = jnp.maximum(w2 + b2b, 0.0)
            p2_ref[pl.ds(80 * pr, 80), :] = w2.reshape(80, NC)

        p2b = p2_ref[...].astype(bf16)
        h1 = jnp.dot(wf1_ref[...].astype(bf16), p2b,
                     preferred_element_type=f32) + bf1_ref[...]
        h1 = jnp.maximum(h1, 0.0).astype(bf16)
        h2 = jnp.dot(wf2_ref[...].astype(bf16), h1,
                     preferred_element_type=f32) + bf2_ref[...]
        h2 = jnp.maximum(h2, 0.0).astype(bf16)
        logits = jnp.dot(wf3_ref[...].astype(bf16), h2,
                         preferred_element_type=f32) + bf3_ref[...]
        o_ref[:, sl] = logits.astype(o_ref.dtype)


def _const_spec(shape):
    return pl.BlockSpec(tuple(shape), lambda b: (0,) * len(shape))


@jax.jit
def _lenet_forward(x, c1w, c1b, c2w, c2b, f1w, f1b, f2w, f2b, f3w, f3b):
    B = x.shape[0]
    f32 = jnp.float32
    bf16 = jnp.bfloat16
    TB = min(2048, B)
    grid_b = pl.cdiv(B, TB)
    Bp = grid_b * TB

    x2 = jnp.transpose(x.astype(f32), (1, 2, 3, 0)).reshape(1024, B)
    if Bp != B:
        x2 = jnp.pad(x2, ((0, 0), (0, Bp - B)))

    c1r = c1w.astype(bf16).reshape(6, 25)
    t1 = jnp.einsum('ct,tpk->pck', c1r, _G1,
                    preferred_element_type=bf16)
    t1 = jnp.pad(t1.reshape(2, 4, 84, 256), ((0, 0), (0, 0), (0, 4), (0, 0)))
    t1 = t1.reshape(704, 256)
    c2r = c2w.astype(bf16).transpose(0, 2, 3, 1).reshape(16, 150)
    t2 = jnp.einsum('ct,tpk->pck', c2r, _G2,
                    preferred_element_type=bf16).reshape(320, 576)

    b1p = jnp.pad(jnp.tile(c1b.astype(f32), 14), (0, 4)).reshape(88, 1)
    b2 = c2b.astype(f32).reshape(16, 1)
    wf1 = (f1w.astype(f32).reshape(120, 16, 5, 5)
           .transpose(0, 2, 3, 1).reshape(120, 400))
    bf1 = f1b.astype(f32).reshape(120, 1)
    wf2 = f2w.astype(f32)
    bf2 = f2b.astype(f32).reshape(84, 1)
    wf3 = f3w.astype(f32)
    bf3 = f3b.astype(f32).reshape(10, 1)

    grid_spec = pltpu.PrefetchScalarGridSpec(
        num_scalar_prefetch=0,
        grid=(grid_b,),
        in_specs=[
            pl.BlockSpec((1024, TB), lambda b: (0, b)),
            _const_spec(t1.shape), _const_spec(b1p.shape),
            _const_spec(t2.shape), _const_spec(b2.shape),
            _const_spec(wf1.shape), _const_spec(bf1.shape),
            _const_spec(wf2.shape), _const_spec(bf2.shape),
            _const_spec(wf3.shape), _const_spec(bf3.shape),
        ],
        out_specs=pl.BlockSpec((10, TB), lambda b: (0, b)),
        scratch_shapes=[
            pltpu.VMEM((1344, 256), bf16),
            pltpu.VMEM((400, 256), f32),
        ],
    )

    out = pl.pallas_call(
        _lenet_body,
        out_shape=jax.ShapeDtypeStruct((10, Bp), f32),
        grid_spec=grid_spec,
        compiler_params=pltpu.CompilerParams(
            dimension_semantics=("parallel",),
            vmem_limit_bytes=48 * 1024 * 1024,
        ),
    )(x2, t1, b1p, t2, b2, wf1, bf1, wf2, bf2, wf3, bf3)

    return out[:, :B].T


def kernel(x, c1w, c1b, c2w, c2b, f1w, f1b, f2w, f2b, f3w, f3b):
    return _lenet_forward(x, c1w, c1b, c2w, c2b,
                          f1w, f1b, f2w, f2b, f3w, f3b)


# speedup vs baseline: 1.3860x; 1.3860x over previous
import ml_dtypes
import numpy as np
import jax
import jax.numpy as jnp
from jax import lax
from jax.experimental import pallas as pl
from jax.experimental.pallas import tpu as pltpu


def _build_g1() -> np.ndarray:
    g = np.zeros((25, 112, 256), np.float32)
    q = np.arange(4)[:, None]
    j = np.arange(28)[None, :]
    p = (28 * q + j).ravel()
    for di in range(5):
        for dj in range(5):
            k = (32 * (q + di) + j + dj).ravel()
            g[5 * di + dj, p, k] = 1.0
    return g


def _build_g2() -> np.ndarray:
    g = np.zeros((150, 20, 504), np.float32)
    rr = np.arange(2)[:, None]
    j = np.arange(10)[None, :]
    p = (10 * rr + j).ravel()
    for di in range(5):
        for dj in range(5):
            for ci in range(6):
                k = (84 * (rr + di) + 6 * (j + dj) + ci).ravel()
                g[(5 * di + dj) * 6 + ci, p, k] = 1.0
    return g


_BF16 = np.dtype(ml_dtypes.bfloat16)
_G1 = _build_g1().astype(_BF16)
_G2 = _build_g2().astype(_BF16)


def _lenet_body(x_ref, t1_ref, b1_ref, t2_ref, b2_ref,
                wf1_ref, bf1_ref, wf2_ref, bf2_ref, wf3_ref, bf3_ref,
                o_ref, p1_ref, p2_ref):
    f32 = jnp.float32
    bf16 = jnp.bfloat16
    TB = x_ref.shape[1]

    t1 = t1_ref[...]
    b1b = b1_ref[...].reshape(1, 6, 1)
    for g in range(7):
        slab = x_ref[pl.ds(128 * g, 256), :].astype(bf16)
        acc = lax.dot_general(t1, slab, (((1,), (0,)), ((), ())),
                              preferred_element_type=f32)
        a4 = acc.reshape(4, 224, TB)
        rows = []
        for half in range(2):
            u = jnp.maximum(a4[2 * half], a4[2 * half + 1])
            v = u.reshape(14, 2, 8, TB)
            w = jnp.maximum(v[:, 0], v[:, 1])
            w = jnp.maximum(w[:, :6, :] + b1b, 0.0)
            rows.append(w.reshape(84, TB))
        p1_ref[pl.ds(168 * g, 168), :] = jnp.concatenate(rows, axis=0)

    t2 = t2_ref[...]
    b2b = b2_ref[...].reshape(1, 16, 1)
    for pr in range(5):
        slab2 = p1_ref[pl.ds(168 * pr, 504), :].astype(bf16)
        acc2 = lax.dot_general(t2, slab2, (((1,), (0,)), ((), ())),
                               preferred_element_type=f32)
        a2 = acc2.reshape(2, 160, TB)
        u2 = jnp.maximum(a2[0], a2[1])
        v2 = u2.reshape(5, 2, 16, TB)
        w2 = jnp.maximum(v2[:, 0], v2[:, 1])
        w2 = jnp.maximum(w2 + b2b, 0.0)
        p2_ref[pl.ds(80 * pr, 80), :] = w2.reshape(80, TB)

    p2b = p2_ref[...].astype(bf16)
    h1 = jnp.dot(wf1_ref[...].astype(bf16), p2b,
                 preferred_element_type=f32) + bf1_ref[...]
    h1 = jnp.maximum(h1, 0.0).astype(bf16)
    h2 = jnp.dot(wf2_ref[...].astype(bf16), h1,
                 preferred_element_type=f32) + bf2_ref[...]
    h2 = jnp.maximum(h2, 0.0).astype(bf16)
    logits = jnp.dot(wf3_ref[...].astype(bf16), h2,
                     preferred_element_type=f32) + bf3_ref[...]
    o_ref[...] = logits.astype(o_ref.dtype)


def _const_spec(shape):
    return pl.BlockSpec(tuple(shape), lambda b: (0,) * len(shape))


@jax.jit
def _lenet_forward(x, c1w, c1b, c2w, c2b, f1w, f1b, f2w, f2b, f3w, f3b):
    B = x.shape[0]
    f32 = jnp.float32
    bf16 = jnp.bfloat16
    TB = min(2048, B)
    grid_b = pl.cdiv(B, TB)
    Bp = grid_b * TB

    x2 = jnp.transpose(x.astype(f32), (1, 2, 3, 0)).reshape(1024, B)
    if Bp != B:
        x2 = jnp.pad(x2, ((0, 0), (0, Bp - B)))

    c1r = jnp.pad(c1w.astype(bf16).reshape(6, 25), ((0, 2), (0, 0)))
    t1 = jnp.einsum('ct,tpk->pck', c1r, _G1,
                    preferred_element_type=bf16).reshape(896, 256)
    c2r = c2w.astype(bf16).transpose(0, 2, 3, 1).reshape(16, 150)
    t2 = jnp.einsum('ct,tpk->pck', c2r, _G2,
                    preferred_element_type=bf16).reshape(320, 504)

    b1p = c1b.astype(f32).reshape(6, 1)
    b2 = c2b.astype(f32).reshape(16, 1)
    wf1 = (f1w.astype(f32).reshape(120, 16, 5, 5)
           .transpose(0, 2, 3, 1).reshape(120, 400))
    bf1 = f1b.astype(f32).reshape(120, 1)
    wf2 = f2w.astype(f32)
    bf2 = f2b.astype(f32).reshape(84, 1)
    wf3 = f3w.astype(f32)
    bf3 = f3b.astype(f32).reshape(10, 1)

    grid_spec = pltpu.PrefetchScalarGridSpec(
        num_scalar_prefetch=0,
        grid=(grid_b,),
        in_specs=[
            pl.BlockSpec((1024, TB), lambda b: (0, b)),
            _const_spec(t1.shape), _const_spec(b1p.shape),
            _const_spec(t2.shape), _const_spec(b2.shape),
            _const_spec(wf1.shape), _const_spec(bf1.shape),
            _const_spec(wf2.shape), _const_spec(bf2.shape),
            _const_spec(wf3.shape), _const_spec(bf3.shape),
        ],
        out_specs=pl.BlockSpec((10, TB), lambda b: (0, b)),
        scratch_shapes=[
            pltpu.VMEM((1176, TB), f32),
            pltpu.VMEM((400, TB), f32),
        ],
    )

    out = pl.pallas_call(
        _lenet_body,
        out_shape=jax.ShapeDtypeStruct((10, Bp), f32),
        grid_spec=grid_spec,
        compiler_params=pltpu.CompilerParams(
            dimension_semantics=("parallel",),
            vmem_limit_bytes=48 * 1024 * 1024,
        ),
    )(x2, t1, b1p, t2, b2, wf1, bf1, wf2, bf2, wf3, bf3)

    return out[:, :B].T


def kernel(x, c1w, c1b, c2w, c2b, f1w, f1b, f2w, f2b, f3w, f3b):
    return _lenet_forward(x, c1w, c1b, c2w, c2b,
                          f1w, f1b, f2w, f2b, f3w, f3b)
